# baseline (speedup 1.0000x reference)
import jax
import jax.numpy as jnp
from jax import lax
from jax.experimental import pallas as pl
from jax.experimental.pallas import tpu as pltpu
from jax.experimental.pallas import tpu_sc as plsc

_B = 8
_N = 100000
_NPAD = 100096
_ROWS = _NPAD // 128
_TOTAL = _B * _NPAD
_K = int(0.8 * (_B * _N))

_NTILES = 32
_CHUNK = _TOTAL // _NTILES
_VECS = _CHUNK // 16
_NBUCKETS = 1 << 15


def _tc1_body(x_ref, t_ref, w_ref, bits_ref, dice_ref):
    row = lax.broadcasted_iota(jnp.int32, (_ROWS, 128), 0)
    col = lax.broadcasted_iota(jnp.int32, (_ROWS, 128), 1)
    valid = (row * 128 + col) < _N
    for b in range(_B):
        x = x_ref[b]
        t = t_ref[b]
        w = w_ref[b]
        e = jnp.exp(-jnp.abs(x))
        bce = (jnp.maximum(x, 0.0) - x * t + jnp.log(1.0 + e)) * w
        bits_ref[b] = lax.bitcast_convert_type(bce, jnp.int32)
        probs = jnp.where(valid, 1.0 / (1.0 + jnp.exp(-x)), 0.0)
        dice_ref[0, b] = jnp.sum(probs * t)
        dice_ref[1, b] = jnp.sum(probs)
        dice_ref[2, b] = jnp.sum(t)


def _sc_hist_body(bits_hbm, hist_hbm, data_v, hist_v):
    c = lax.axis_index("c")
    s = lax.axis_index("s")
    wid = c * 16 + s
    pltpu.sync_copy(bits_hbm.at[pl.ds(wid * _CHUNK, _CHUNK)], data_v)

    def zero_body(j, carry):
        hist_v[pl.ds(j * 16, 16)] = jnp.zeros((16,), jnp.int32)
        return carry

    lax.fori_loop(0, _NBUCKETS // 16, zero_body, 0)

    def hist_body(i, carry):
        v = data_v[pl.ds(i * 16, 16)]
        idx = lax.shift_right_logical(v, 16)
        cnts, last = plsc.scan_count(idx)
        plsc.addupdate_scatter(hist_v, [idx], cnts, mask=last)
        return carry

    lax.fori_loop(0, _VECS, hist_body, 0)
    pltpu.sync_copy(hist_v, hist_hbm.at[wid])


def _tc2_body(bits_ref, hists_ref, out_ref):
    hist = hists_ref[0]
    for i in range(1, _NTILES):
        hist = hist + hists_ref[i]
    histf = hist.astype(jnp.float32)
    r = lax.broadcasted_iota(jnp.int32, (_NBUCKETS // 128, 128), 0)
    cc = lax.broadcasted_iota(jnp.int32, (_NBUCKETS // 128, 128), 1)
    bidx = r * 128 + cc
    kf = jnp.float32(_K)

    def search(i, tb):
        cand = tb | (jnp.int32(1) << (jnp.int32(14) - i))
        cnt = jnp.sum(jnp.where(bidx >= cand, histf, 0.0))
        return jnp.where(cnt >= kf, cand, tb)

    tbucket = lax.fori_loop(0, 15, search, jnp.int32(0))

    s_hi = jnp.float32(0.0)
    c_hi = jnp.float32(0.0)
    s_eq = jnp.float32(0.0)
    c_eq = jnp.float32(0.0)
    for b in range(_B):
        bits = bits_ref[b]
        vals = lax.bitcast_convert_type(bits, jnp.float32)
        b15 = lax.shift_right_logical(bits, 16)
        hi = b15 > tbucket
        eq = b15 == tbucket
        s_hi += jnp.sum(jnp.where(hi, vals, 0.0))
        c_hi += jnp.sum(jnp.where(hi, 1.0, 0.0))
        s_eq += jnp.sum(jnp.where(eq, vals, 0.0))
        c_eq += jnp.sum(jnp.where(eq, 1.0, 0.0))
    out_ref[0, 0] = s_hi
    out_ref[0, 1] = c_hi
    out_ref[0, 2] = s_eq
    out_ref[0, 3] = c_eq


def kernel(pred_logits, target_labels, weights):
    def prep(a):
        a = a.reshape(_B, _N)
        a = jnp.pad(a, ((0, 0), (0, _NPAD - _N)))
        return a.reshape(_B, _ROWS, 128)

    x, t, w = prep(pred_logits), prep(target_labels), prep(weights)

    bits, dice = pl.pallas_call(
        _tc1_body,
        out_shape=(
            jax.ShapeDtypeStruct((_B, _ROWS, 128), jnp.int32),
            jax.ShapeDtypeStruct((3, _B), jnp.float32),
        ),
        out_specs=(
            pl.BlockSpec(memory_space=pltpu.VMEM),
            pl.BlockSpec(memory_space=pltpu.SMEM),
        ),
    )(x, t, w)

    sc_hist = pl.kernel(
        _sc_hist_body,
        out_type=jax.ShapeDtypeStruct((_NTILES, _NBUCKETS), jnp.int32),
        mesh=plsc.VectorSubcoreMesh(
            core_axis_name="c", subcore_axis_name="s", num_cores=2, num_subcores=16
        ),
        scratch_types=[
            pltpu.VMEM((_CHUNK,), jnp.int32),
            pltpu.VMEM((_NBUCKETS,), jnp.int32),
        ],
        compiler_params=pltpu.CompilerParams(needs_layout_passes=False),
    )
    hists = sc_hist(bits.reshape(_TOTAL))

    stats = pl.pallas_call(
        _tc2_body,
        out_shape=jax.ShapeDtypeStruct((1, 4), jnp.float32),
        out_specs=pl.BlockSpec(memory_space=pltpu.SMEM),
    )(bits, hists.reshape(_NTILES, _NBUCKETS // 128, 128))

    s_hi, c_hi, s_eq, c_eq = stats[0, 0], stats[0, 1], stats[0, 2], stats[0, 3]
    bucket_mean = s_eq / jnp.maximum(c_eq, 1.0)
    s_top = s_hi + (jnp.float32(_K) - c_hi) * bucket_mean
    bce_loss = s_top / jnp.float32(_K)

    inter, sum_p, sum_t = dice[0], dice[1], dice[2]
    dice_score = (2.0 * inter + 1e-06) / (sum_p + sum_t + 1e-06)
    dice_loss = jnp.mean(jnp.log(jnp.cosh(1.0 - dice_score)))
    total = 1.0 * bce_loss + 10.0 * dice_loss
    return (total, lax.stop_gradient(bce_loss), lax.stop_gradient(dice_loss))


# speedup vs baseline: 1.4515x; 1.4515x over previous
import jax
import jax.numpy as jnp
from jax import lax
from jax.experimental import pallas as pl
from jax.experimental.pallas import tpu as pltpu
from jax.experimental.pallas import tpu_sc as plsc

_B = 8
_N = 100000
_NPAD = 100096
_ROWS = _NPAD // 128
_TOTAL = _B * _NPAD
_K = int(0.8 * (_B * _N))

_NTILES = 32
_CHUNK = _TOTAL // _NTILES
_VECS = _CHUNK // 16
_NBUCKETS = 1 << 15


def _tc1_body(x_ref, t_ref, w_ref, bits_ref, dice_ref):
    row = lax.broadcasted_iota(jnp.int32, (_ROWS, 128), 0)
    col = lax.broadcasted_iota(jnp.int32, (_ROWS, 128), 1)
    valid = (row * 128 + col) < _N
    for b in range(_B):
        x = x_ref[b]
        t = t_ref[b]
        w = w_ref[b]
        e = jnp.exp(-jnp.abs(x))
        bce = (jnp.maximum(x, 0.0) - x * t + jnp.log(1.0 + e)) * w
        bits_ref[b] = lax.bitcast_convert_type(bce, jnp.int32)
        probs = jnp.where(valid, 1.0 / (1.0 + jnp.exp(-x)), 0.0)
        dice_ref[0, b] = jnp.sum(probs * t)
        dice_ref[1, b] = jnp.sum(probs)
        dice_ref[2, b] = jnp.sum(t)


def _sc_hist_body(bits_hbm, hist_hbm, data_v, hist_v):
    c = lax.axis_index("c")
    s = lax.axis_index("s")
    wid = c * 16 + s
    pltpu.sync_copy(bits_hbm.at[pl.ds(wid * _CHUNK, _CHUNK)], data_v)

    @plsc.parallel_loop(0, _NBUCKETS // 16, unroll=8)
    def _zero_body(j):
        hist_v[pl.ds(j * 16, 16)] = jnp.zeros((16,), jnp.int32)

    @plsc.parallel_loop(0, _VECS, unroll=4)
    def _hist_body(i):
        v = data_v[pl.ds(i * 16, 16)]
        idx = lax.shift_right_logical(v, 16)
        cnts, last = plsc.scan_count(idx)
        plsc.addupdate_scatter(hist_v, [idx], cnts, mask=last)

    pltpu.sync_copy(hist_v, hist_hbm.at[wid])


def _tc2_body(bits_ref, hists_ref, out_ref):
    hist = hists_ref[0]
    for i in range(1, _NTILES):
        hist = hist + hists_ref[i]
    histf = hist.astype(jnp.float32)
    r = lax.broadcasted_iota(jnp.int32, (_NBUCKETS // 128, 128), 0)
    cc = lax.broadcasted_iota(jnp.int32, (_NBUCKETS // 128, 128), 1)
    bidx = r * 128 + cc
    kf = jnp.float32(_K)

    def search(i, tb):
        cand = tb | (jnp.int32(1) << (jnp.int32(14) - i))
        cnt = jnp.sum(jnp.where(bidx >= cand, histf, 0.0))
        return jnp.where(cnt >= kf, cand, tb)

    tbucket = lax.fori_loop(0, 15, search, jnp.int32(0))

    s_hi = jnp.float32(0.0)
    c_hi = jnp.float32(0.0)
    s_eq = jnp.float32(0.0)
    c_eq = jnp.float32(0.0)
    for b in range(_B):
        bits = bits_ref[b]
        vals = lax.bitcast_convert_type(bits, jnp.float32)
        b15 = lax.shift_right_logical(bits, 16)
        hi = b15 > tbucket
        eq = b15 == tbucket
        s_hi += jnp.sum(jnp.where(hi, vals, 0.0))
        c_hi += jnp.sum(jnp.where(hi, 1.0, 0.0))
        s_eq += jnp.sum(jnp.where(eq, vals, 0.0))
        c_eq += jnp.sum(jnp.where(eq, 1.0, 0.0))
    out_ref[0, 0] = s_hi
    out_ref[0, 1] = c_hi
    out_ref[0, 2] = s_eq
    out_ref[0, 3] = c_eq


def kernel(pred_logits, target_labels, weights):
    def prep(a):
        a = a.reshape(_B, _N)
        a = jnp.pad(a, ((0, 0), (0, _NPAD - _N)))
        return a.reshape(_B, _ROWS, 128)

    x, t, w = prep(pred_logits), prep(target_labels), prep(weights)

    bits, dice = pl.pallas_call(
        _tc1_body,
        out_shape=(
            jax.ShapeDtypeStruct((_B, _ROWS, 128), jnp.int32),
            jax.ShapeDtypeStruct((3, _B), jnp.float32),
        ),
        out_specs=(
            pl.BlockSpec(memory_space=pltpu.VMEM),
            pl.BlockSpec(memory_space=pltpu.SMEM),
        ),
    )(x, t, w)

    sc_hist = pl.kernel(
        _sc_hist_body,
        out_type=jax.ShapeDtypeStruct((_NTILES, _NBUCKETS), jnp.int32),
        mesh=plsc.VectorSubcoreMesh(
            core_axis_name="c", subcore_axis_name="s", num_cores=2, num_subcores=16
        ),
        scratch_types=[
            pltpu.VMEM((_CHUNK,), jnp.int32),
            pltpu.VMEM((_NBUCKETS,), jnp.int32),
        ],
        compiler_params=pltpu.CompilerParams(needs_layout_passes=False),
    )
    hists = sc_hist(bits.reshape(_TOTAL))

    stats = pl.pallas_call(
        _tc2_body,
        out_shape=jax.ShapeDtypeStruct((1, 4), jnp.float32),
        out_specs=pl.BlockSpec(memory_space=pltpu.SMEM),
    )(bits, hists.reshape(_NTILES, _NBUCKETS // 128, 128))

    s_hi, c_hi, s_eq, c_eq = stats[0, 0], stats[0, 1], stats[0, 2], stats[0, 3]
    bucket_mean = s_eq / jnp.maximum(c_eq, 1.0)
    s_top = s_hi + (jnp.float32(_K) - c_hi) * bucket_mean
    bce_loss = s_top / jnp.float32(_K)

    inter, sum_p, sum_t = dice[0], dice[1], dice[2]
    dice_score = (2.0 * inter + 1e-06) / (sum_p + sum_t + 1e-06)
    dice_loss = jnp.mean(jnp.log(jnp.cosh(1.0 - dice_score)))
    total = 1.0 * bce_loss + 10.0 * dice_loss
    return (total, lax.stop_gradient(bce_loss), lax.stop_gradient(dice_loss))
